# baseline (speedup 1.0000x reference)
import numpy as np
import jax
import jax.numpy as jnp
from jax.experimental import pallas as pl

_ROWS = 128
_VOCAB = 100000
_R = 8

_NOISE = np.maximum(
    np.asarray(
        jax.random.exponential(jax.random.key(42), (_ROWS, _VOCAB), jnp.float32)
    ),
    np.float32(1e-10),
)
_NEG_LOG_NOISE = -np.log(_NOISE)
_INV_NOISE = (np.float32(1.0) / _NOISE).astype(np.float32)


def _top2_kernel(logits_ref, gumbel_ref, temp_ref, i1_ref, i2_ref):
    x = logits_ref[...]
    t = temp_ref[...]
    inv_t = 1.0 / jnp.maximum(t, 1e-10)
    v = x * inv_t + gumbel_ref[...]
    i1 = jnp.argmax(v, axis=-1)
    iota = jax.lax.broadcasted_iota(jnp.int32, v.shape, 1)
    v2 = jnp.where(iota == i1[:, None], -jnp.inf, v)
    i2 = jnp.argmax(v2, axis=-1)
    i1_ref[...] = i1[:, None]
    i2_ref[...] = i2[:, None]


def kernel(logits, temperatures):
    logits = logits.astype(jnp.float32)
    gumbel = jnp.asarray(_NEG_LOG_NOISE)
    temps = temperatures.astype(jnp.float32).reshape(_ROWS, 1)
    i1, i2 = pl.pallas_call(
        _top2_kernel,
        grid=(_ROWS // _R,),
        in_specs=[
            pl.BlockSpec((_R, _VOCAB), lambda i: (i, 0)),
            pl.BlockSpec((_R, _VOCAB), lambda i: (i, 0)),
            pl.BlockSpec((_R, 1), lambda i: (i, 0)),
        ],
        out_specs=[
            pl.BlockSpec((_R, 1), lambda i: (i, 0)),
            pl.BlockSpec((_R, 1), lambda i: (i, 0)),
        ],
        out_shape=[
            jax.ShapeDtypeStruct((_ROWS, 1), jnp.int32),
            jax.ShapeDtypeStruct((_ROWS, 1), jnp.int32),
        ],
    )(logits, gumbel, temps)

    cand = jnp.concatenate([i1, i2], axis=1)
    xg = jnp.take_along_axis(logits, cand, axis=1)
    ng = jnp.take_along_axis(jnp.asarray(_INV_NOISE), cand, axis=1)
    safe_t = jnp.maximum(temps, 1e-10)
    w = xg / safe_t
    r = jnp.exp(w - jnp.max(w, axis=1, keepdims=True)) * ng
    pick = jnp.argmax(r, axis=1)
    sample = jnp.take_along_axis(cand, pick[:, None], axis=1)[:, 0]

    g_hi = jnp.where(
        xg[:, 0] > xg[:, 1],
        cand[:, 0],
        jnp.where(xg[:, 1] > xg[:, 0], cand[:, 1], jnp.minimum(cand[:, 0], cand[:, 1])),
    )
    t_flat = temps[:, 0]
    return jnp.where(t_flat <= 1e-10, g_hi, sample)


# speedup vs baseline: 1.0886x; 1.0886x over previous
import numpy as np
import jax
import jax.numpy as jnp
from jax.experimental import pallas as pl

_ROWS = 128
_VOCAB = 100000
_R = 8


def _rotl32(x, d):
    return (x << np.uint32(d)) | (x >> np.uint32(32 - d))


def _threefry2x32(k0, k1, x0, x1):
    ks0, ks1 = np.uint32(k0), np.uint32(k1)
    ks2 = ks0 ^ ks1 ^ np.uint32(0x1BD11BDA)
    rot = ((13, 15, 26, 6), (17, 29, 16, 24))
    ks = (ks0, ks1, ks2)
    x0 = x0 + ks0
    x1 = x1 + ks1
    for i in range(5):
        for r in rot[i % 2]:
            x0 = x0 + x1
            x1 = _rotl32(x1, r)
            x1 = x0 ^ x1
        x0 = x0 + ks[(i + 1) % 3]
        x1 = x1 + ks[(i + 2) % 3] + np.uint32(i + 1)
    return x0, x1


def _exponential_noise(shape, seed=42):
    n = int(np.prod(shape))
    idx = np.arange(n, dtype=np.uint64)
    c1 = (idx >> np.uint64(32)).astype(np.uint32)
    c2 = (idx & np.uint64(0xFFFFFFFF)).astype(np.uint32)
    b1, b2 = _threefry2x32(np.uint32(0), np.uint32(seed), c1, c2)
    bits = b1 ^ b2
    fb = (bits >> np.uint32(9)) | np.uint32(0x3F800000)
    u = fb.view(np.float32) - np.float32(1.0)
    return (-np.log1p(-u)).reshape(shape)


_INV_NOISE = (
    np.float32(1.0)
    / np.maximum(_exponential_noise((_ROWS, _VOCAB)), np.float32(1e-10))
).astype(np.float32)


def _sample_kernel(logits_ref, inv_noise_ref, temp_ref, out_ref):
    x = logits_ref[...]
    t = temp_ref[...]
    safe_t = jnp.maximum(t, 1e-10)
    s = x / safe_t
    greedy = jnp.argmax(s, axis=-1)
    m = jnp.max(s, axis=-1, keepdims=True)
    r = jnp.exp(s - m) * inv_noise_ref[...]
    sample = jnp.argmax(r, axis=-1)
    tok = jnp.where(t[:, 0] <= 1e-10, greedy, sample)
    out_ref[...] = tok[:, None]


def kernel(logits, temperatures):
    logits = logits.astype(jnp.float32)
    inv_noise = jnp.asarray(_INV_NOISE)
    temps = temperatures.astype(jnp.float32).reshape(_ROWS, 1)
    out = pl.pallas_call(
        _sample_kernel,
        grid=(_ROWS // _R,),
        in_specs=[
            pl.BlockSpec((_R, _VOCAB), lambda i: (i, 0)),
            pl.BlockSpec((_R, _VOCAB), lambda i: (i, 0)),
            pl.BlockSpec((_R, 1), lambda i: (i, 0)),
        ],
        out_specs=pl.BlockSpec((_R, 1), lambda i: (i, 0)),
        out_shape=jax.ShapeDtypeStruct((_ROWS, 1), jnp.int32),
    )(logits, inv_noise, temps)
    return out.reshape(_ROWS)


# speedup vs baseline: 1.1210x; 1.0297x over previous
import numpy as np
import jax
import jax.numpy as jnp
from jax.experimental import pallas as pl

_ROWS = 128
_VOCAB = 100000
_R = 16


def _rotl32(x, d):
    return (x << np.uint32(d)) | (x >> np.uint32(32 - d))


def _threefry2x32(k0, k1, x0, x1):
    ks0, ks1 = np.uint32(k0), np.uint32(k1)
    ks2 = ks0 ^ ks1 ^ np.uint32(0x1BD11BDA)
    rot = ((13, 15, 26, 6), (17, 29, 16, 24))
    ks = (ks0, ks1, ks2)
    x0 = x0 + ks0
    x1 = x1 + ks1
    for i in range(5):
        for r in rot[i % 2]:
            x0 = x0 + x1
            x1 = _rotl32(x1, r)
            x1 = x0 ^ x1
        x0 = x0 + ks[(i + 1) % 3]
        x1 = x1 + ks[(i + 2) % 3] + np.uint32(i + 1)
    return x0, x1


def _exponential_noise(shape, seed=42):
    n = int(np.prod(shape))
    idx = np.arange(n, dtype=np.uint64)
    c1 = (idx >> np.uint64(32)).astype(np.uint32)
    c2 = (idx & np.uint64(0xFFFFFFFF)).astype(np.uint32)
    b1, b2 = _threefry2x32(np.uint32(0), np.uint32(seed), c1, c2)
    bits = b1 ^ b2
    fb = (bits >> np.uint32(9)) | np.uint32(0x3F800000)
    u = fb.view(np.float32) - np.float32(1.0)
    return (-np.log1p(-u)).reshape(shape)


_INV_NOISE = (
    np.float32(1.0)
    / np.maximum(_exponential_noise((_ROWS, _VOCAB)), np.float32(1e-10))
).astype(np.float32)


def _sample_kernel(logits_ref, inv_noise_ref, temp_ref, out_ref):
    x = logits_ref[...]
    t = temp_ref[...]
    safe_t = jnp.maximum(t, 1e-10)
    s = x / safe_t
    greedy = jnp.argmax(s, axis=-1)
    m = jnp.max(s, axis=-1, keepdims=True)
    r = jnp.exp(s - m) * inv_noise_ref[...]
    sample = jnp.argmax(r, axis=-1)
    tok = jnp.where(t[:, 0] <= 1e-10, greedy, sample)
    out_ref[...] = tok[:, None]


def kernel(logits, temperatures):
    logits = logits.astype(jnp.float32)
    inv_noise = jnp.asarray(_INV_NOISE)
    temps = temperatures.astype(jnp.float32).reshape(_ROWS, 1)
    out = pl.pallas_call(
        _sample_kernel,
        grid=(_ROWS // _R,),
        in_specs=[
            pl.BlockSpec((_R, _VOCAB), lambda i: (i, 0)),
            pl.BlockSpec((_R, _VOCAB), lambda i: (i, 0)),
            pl.BlockSpec((_R, 1), lambda i: (i, 0)),
        ],
        out_specs=pl.BlockSpec((_R, 1), lambda i: (i, 0)),
        out_shape=jax.ShapeDtypeStruct((_ROWS, 1), jnp.int32),
    )(logits, inv_noise, temps)
    return out.reshape(_ROWS)
